# baseline (speedup 1.0000x reference)
import functools

import jax
import jax.numpy as jnp
from jax import lax
from jax.experimental import pallas as pl
from jax.experimental.pallas import tpu as pltpu
from jax.experimental.pallas import tpu_sc as plsc

N = 10000
E = 320000
D = 128
NC = 2
NS = 16
NW = NC * NS
CH = 125
NCHT = E // CH
NCHUNK = NCHT // NW
UNROLL = 4
NITER = NCHUNK // UNROLL
RPT = 632
RPT_LAST = N - (NS - 1) * RPT
N2 = 10240
DPT = N2 // NS

_mesh = plsc.VectorSubcoreMesh(
    core_axis_name="c", subcore_axis_name="s", num_cores=NC, num_subcores=NS
)


CHD = 128
NCHD_TOT = E // CHD
NCHD = (NCHD_TOT + NW - 1) // NW


@functools.partial(
    pl.kernel,
    out_type=jax.ShapeDtypeStruct((NC, N2), jnp.float32),
    mesh=_mesh,
    compiler_params=pltpu.CompilerParams(needs_layout_passes=False),
    scratch_types=[
        pltpu.VMEM((N2,), jnp.float32),
        pltpu.VMEM((2, CHD), jnp.int32),
        pltpu.VMEM((NS, DPT), jnp.float32),
        pltpu.VMEM_SHARED((NS, N2), jnp.float32),
        pltpu.SemaphoreType.DMA,
    ],
)
def _deg_kernel(ei_hbm, out_hbm, hist_v, idx_v, mrg_v, slab_sh, isem):
    c = lax.axis_index("c")
    s = lax.axis_index("s")
    w = c * NS + s

    def zero(i, carry):
        hist_v[pl.ds(i * 16, 16)] = jnp.zeros((16,), jnp.float32)
        return carry

    lax.fori_loop(0, N2 // 16, zero, 0)

    pltpu.async_copy(ei_hbm.at[1, pl.ds(w * CHD, CHD)], idx_v.at[0], isem).wait()

    ones16 = jnp.full((16,), 1.0, jnp.float32)

    def accum(r):
        for i in range(CHD // 16):
            idx16 = idx_v[r, pl.ds(i * 16, 16)]
            plsc.addupdate_scatter(hist_v, [idx16], ones16)

    def body(k, carry):
        r = lax.rem(k, 2)
        r1 = lax.rem(k + 1, 2)
        nxt = w + (k + 1) * NW

        @pl.when(nxt < NCHD_TOT)
        def _():
            d = pltpu.async_copy(
                ei_hbm.at[1, pl.ds(nxt * CHD, CHD)], idx_v.at[r1], isem
            )
            accum(r)
            d.wait()

        @pl.when(jnp.logical_and(nxt >= NCHD_TOT, w + k * NW < NCHD_TOT))
        def _():
            accum(r)

        return carry

    lax.fori_loop(0, NCHD, body, 0)

    pltpu.sync_copy(hist_v, slab_sh.at[s])
    plsc.subcore_barrier()
    pltpu.sync_copy(slab_sh.at[:, pl.ds(s * DPT, DPT)], mrg_v)

    def merge(i, carry):
        acc = jnp.full((16,), 0.5, jnp.float32)
        for t in range(NS):
            acc = acc + mrg_v[t, pl.ds(i * 16, 16)]
        hist_v[pl.ds(i * 16, 16)] = acc
        return carry

    lax.fori_loop(0, DPT // 16, merge, 0)
    pltpu.sync_copy(
        hist_v.at[pl.ds(0, DPT)], out_hbm.at[c, pl.ds(s * DPT, DPT)]
    )


@functools.partial(
    pl.kernel,
    out_type=jax.ShapeDtypeStruct((NC, N, D), jnp.float32),
    mesh=_mesh,
    scratch_types=[
        pltpu.VMEM((UNROLL, 2, CH), jnp.int32),
        pltpu.VMEM((3, CH, D), jnp.float32),
        pltpu.VMEM_SHARED((N, D), jnp.float32),
        pltpu.SemaphoreType.DMA,
        pltpu.SemaphoreType.DMA,
        pltpu.SemaphoreType.DMA,
        pltpu.SemaphoreType.DMA,
    ],
)
def _edge_kernel(
    h_hbm, idx_hbm, out_hbm, idx_v, rows_v, acc_sh, isem, gsemA, gsemB, asem
):
    c = lax.axis_index("c")
    s = lax.axis_index("s")
    w = c * NS + s
    base = w * NCHUNK

    def prologue(row0, nrows):
        ainit = pltpu.async_copy(
            h_hbm.at[pl.ds(row0, nrows)], acc_sh.at[pl.ds(row0, nrows)], asem
        )
        ip = pltpu.async_copy(
            idx_hbm.at[pl.ds(base, 3)], idx_v.at[pl.ds(0, 3)], isem
        )
        ip.wait()
        g0 = pltpu.async_copy(h_hbm.at[idx_v.at[0, 0]], rows_v.at[0], gsemA)
        pltpu.async_copy(h_hbm.at[idx_v.at[1, 0]], rows_v.at[1], gsemB)
        g0.wait()
        ainit.wait()


    pl.when(s < NS - 1)(lambda: prologue(s * RPT, RPT))
    pl.when(s == NS - 1)(lambda: prologue((NS - 1) * RPT, RPT_LAST))

    plsc.subcore_barrier()

    def body(t, carry):
        for u in range(UNROLL):
            k = t * UNROLL + u
            q, q1, q2, q3 = u, (u + 1) % 4, (u + 2) % 4, (u + 3) % 4
            r = lax.rem(k, 3)
            r1 = lax.rem(k + 1, 3)
            r2 = lax.rem(k + 2, 3)
            gs_issue = (gsemA, gsemB)[u % 2]
            gs_wait = (gsemA, gsemB)[(u + 1) % 2]

            def chunk(do_pf, do_g2, do_w1):
                if do_pf:
                    dpf = pltpu.async_copy(
                        idx_hbm.at[base + k + 3], idx_v.at[q3], isem
                    )
                if do_g2:
                    pltpu.async_copy(
                        h_hbm.at[idx_v.at[q2, 0]], rows_v.at[r2], gs_issue
                    )
                pltpu.sync_copy(rows_v.at[r], acc_sh.at[idx_v.at[q, 1]], add=True)
                if do_w1:
                    pltpu.make_async_copy(
                        h_hbm.at[idx_v.at[q1, 0]], rows_v.at[r1], gs_wait
                    ).wait()
                if do_pf:
                    dpf.wait()

            if u == 0:
                chunk(True, True, True)
            else:
                tails = {1: (False, True, True), 2: (False, False, True),
                         3: (False, False, False)}[u]
                pl.when(t < NITER - 1)(lambda: chunk(True, True, True))
                pl.when(t == NITER - 1)(lambda: chunk(*tails))
        return carry

    lax.fori_loop(0, NITER, body, 0)

    plsc.subcore_barrier()

    @pl.when(s < NS - 1)
    def _():
        pltpu.sync_copy(
            acc_sh.at[pl.ds(s * RPT, RPT)], out_hbm.at[c, pl.ds(s * RPT, RPT)]
        )

    @pl.when(s == NS - 1)
    def _():
        pltpu.sync_copy(
            acc_sh.at[pl.ds((NS - 1) * RPT, RPT_LAST)],
            out_hbm.at[c, pl.ds((NS - 1) * RPT, RPT_LAST)],
        )


_RB = 1000


def _mm0_body(x_ref, w_ref, u_ref):
    u_ref[...] = jnp.dot(x_ref[...], w_ref[...], preferred_element_type=jnp.float32)


def _scale_body(u_ref, dg_ref, h_ref, dinv_ref):
    dinv = lax.rsqrt(dg_ref[:, 0:1] + dg_ref[:, 1:2])
    h_ref[...] = u_ref[...] * dinv
    dinv_ref[...] = dinv


def _mid_body(acc_ref, hp_ref, dinv_ref, b_ref, w_ref, out_ref):
    z = (
        dinv_ref[...] * (acc_ref[0] + acc_ref[1] - hp_ref[...]) + b_ref[...]
    )
    z = jnp.maximum(z, 0.0)
    out_ref[...] = (
        jnp.dot(z, w_ref[...], preferred_element_type=jnp.float32) * dinv_ref[...]
    )


def _fin_body(acc_ref, hp_ref, dinv_ref, b_ref, out_ref):
    out_ref[...] = (
        dinv_ref[...] * (acc_ref[0] + acc_ref[1] - hp_ref[...]) + b_ref[...]
    )


def _row_spec(width):
    return pl.BlockSpec((_RB, width), lambda i: (i, 0))


def _acc_spec():
    return pl.BlockSpec((2, _RB, D), lambda i: (0, i, 0))


def _const_spec(shape):
    return pl.BlockSpec(shape, lambda i: (0, 0))


_mm0 = pl.pallas_call(
    _mm0_body,
    grid=(N // _RB,),
    in_specs=[_row_spec(D), _const_spec((D, D))],
    out_specs=_row_spec(D),
    out_shape=jax.ShapeDtypeStruct((N, D), jnp.float32),
)

_scale = pl.pallas_call(
    _scale_body,
    grid=(N // _RB,),
    in_specs=[_row_spec(D), _row_spec(2)],
    out_specs=[_row_spec(D), _row_spec(1)],
    out_shape=[
        jax.ShapeDtypeStruct((N, D), jnp.float32),
        jax.ShapeDtypeStruct((N, 1), jnp.float32),
    ],
)

_mid = pl.pallas_call(
    _mid_body,
    grid=(N // _RB,),
    in_specs=[
        _acc_spec(), _row_spec(D), _row_spec(1),
        _const_spec((1, D)), _const_spec((D, D)),
    ],
    out_specs=_row_spec(D),
    out_shape=jax.ShapeDtypeStruct((N, D), jnp.float32),
)

_fin = pl.pallas_call(
    _fin_body,
    grid=(N // _RB,),
    in_specs=[
        _acc_spec(), _row_spec(D), _row_spec(1), _const_spec((1, D)),
    ],
    out_specs=_row_spec(D),
    out_shape=jax.ShapeDtypeStruct((N, D), jnp.float32),
)


def kernel(x, edge_index, W1, b1, W2, b2):
    ei = edge_index.astype(jnp.int32)
    idx3 = ei.reshape(2, NCHT, CH).transpose(1, 0, 2)

    deg2 = _deg_kernel(ei)
    u1 = _mm0(x, W1)

    h1p, dinv = _scale(u1, deg2.T)
    acc1 = _edge_kernel(h1p, idx3)
    h2p = _mid(acc1, h1p, dinv, b1.reshape(1, D), W2)
    acc2 = _edge_kernel(h2p, idx3)
    return _fin(acc2, h2p, dinv, b2.reshape(1, D))


# speedup vs baseline: 1.0887x; 1.0887x over previous
import functools

import jax
import jax.numpy as jnp
from jax import lax
from jax.experimental import pallas as pl
from jax.experimental.pallas import tpu as pltpu
from jax.experimental.pallas import tpu_sc as plsc

N = 10000
E = 320000
D = 128
NC = 2
NS = 16
NW = NC * NS
CH = 125
NCHT = E // CH
NCHUNK = NCHT // NW
UNROLL = 4
NITER = NCHUNK // UNROLL
RPT = 632
RPT_LAST = N - (NS - 1) * RPT
N2 = 10240
DPT = N2 // NS

_mesh = plsc.VectorSubcoreMesh(
    core_axis_name="c", subcore_axis_name="s", num_cores=NC, num_subcores=NS
)


CHD = 128
NCHD_TOT = E // CHD
NCHD = (NCHD_TOT + NW - 1) // NW


@functools.partial(
    pl.kernel,
    out_type=jax.ShapeDtypeStruct((NC, N2), jnp.float32),
    mesh=_mesh,
    compiler_params=pltpu.CompilerParams(needs_layout_passes=False),
    scratch_types=[
        pltpu.VMEM((N2,), jnp.float32),
        pltpu.VMEM((4, CHD), jnp.int32),
        pltpu.VMEM((NS, DPT), jnp.float32),
        pltpu.VMEM_SHARED((NS, N2), jnp.float32),
        pltpu.SemaphoreType.DMA,
        pltpu.SemaphoreType.DMA,
        pltpu.SemaphoreType.DMA,
        pltpu.SemaphoreType.DMA,
    ],
)
def _deg_kernel(ei_hbm, out_hbm, hist_v, idx_v, mrg_v, slab_sh, i0, i1, i2, i3):
    c = lax.axis_index("c")
    s = lax.axis_index("s")
    w = c * NS + s
    isems = (i0, i1, i2, i3)

    def zero(i, carry):
        for u in range(4):
            hist_v[pl.ds(i * 64 + u * 16, 16)] = jnp.zeros((16,), jnp.float32)
        return carry

    lax.fori_loop(0, N2 // 64, zero, 0)

    for q in range(3):

        def prime(q=q):
            pltpu.async_copy(
                ei_hbm.at[1, pl.ds((w + q * NW) * CHD, CHD)], idx_v.at[q], isems[q]
            )

        pl.when(w + q * NW < NCHD_TOT)(prime)

    ones16 = jnp.full((16,), 1.0, jnp.float32)

    def accum(q):
        for i in range(CHD // 16):
            idx16 = idx_v[q, pl.ds(i * 16, 16)]
            plsc.addupdate_scatter(hist_v, [idx16], ones16)

    def body(t, carry):
        for u in range(4):
            k = t * 4 + u
            q3 = (u + 3) % 4

            def sub(k=k, u=u, q3=q3):
                pltpu.make_async_copy(
                    ei_hbm.at[1, pl.ds(0, CHD)], idx_v.at[u], isems[u]
                ).wait()

                @pl.when(w + (k + 3) * NW < NCHD_TOT)
                def _():
                    pltpu.async_copy(
                        ei_hbm.at[1, pl.ds((w + (k + 3) * NW) * CHD, CHD)],
                        idx_v.at[q3],
                        isems[q3],
                    )

                accum(u)

            pl.when(w + k * NW < NCHD_TOT)(sub)
        return carry

    lax.fori_loop(0, (NCHD + 3) // 4, body, 0)

    pltpu.sync_copy(hist_v, slab_sh.at[s])
    plsc.subcore_barrier()
    pltpu.sync_copy(slab_sh.at[:, pl.ds(s * DPT, DPT)], mrg_v)

    def merge(i, carry):
        acc = jnp.full((16,), 0.5, jnp.float32)
        for t in range(NS):
            acc = acc + mrg_v[t, pl.ds(i * 16, 16)]
        hist_v[pl.ds(i * 16, 16)] = acc
        return carry

    lax.fori_loop(0, DPT // 16, merge, 0)
    pltpu.sync_copy(
        hist_v.at[pl.ds(0, DPT)], out_hbm.at[c, pl.ds(s * DPT, DPT)]
    )


@functools.partial(
    pl.kernel,
    out_type=jax.ShapeDtypeStruct((NC, N, D), jnp.float32),
    mesh=_mesh,
    scratch_types=[
        pltpu.VMEM((UNROLL, 2, CH), jnp.int32),
        pltpu.VMEM((3, CH, D), jnp.float32),
        pltpu.VMEM_SHARED((N, D), jnp.float32),
        pltpu.SemaphoreType.DMA,
        pltpu.SemaphoreType.DMA,
        pltpu.SemaphoreType.DMA,
        pltpu.SemaphoreType.DMA,
    ],
)
def _edge_kernel(
    h_hbm, idx_hbm, out_hbm, idx_v, rows_v, acc_sh, isem, gsemA, gsemB, asem
):
    c = lax.axis_index("c")
    s = lax.axis_index("s")
    w = c * NS + s
    base = w * NCHUNK

    def prologue(row0, nrows):
        ainit = pltpu.async_copy(
            h_hbm.at[pl.ds(row0, nrows)], acc_sh.at[pl.ds(row0, nrows)], asem
        )
        ip = pltpu.async_copy(
            idx_hbm.at[pl.ds(base, 3)], idx_v.at[pl.ds(0, 3)], isem
        )
        ip.wait()
        g0 = pltpu.async_copy(h_hbm.at[idx_v.at[0, 0]], rows_v.at[0], gsemA)
        pltpu.async_copy(h_hbm.at[idx_v.at[1, 0]], rows_v.at[1], gsemB)
        g0.wait()
        ainit.wait()


    pl.when(s < NS - 1)(lambda: prologue(s * RPT, RPT))
    pl.when(s == NS - 1)(lambda: prologue((NS - 1) * RPT, RPT_LAST))

    plsc.subcore_barrier()

    def body(t, carry):
        for u in range(UNROLL):
            k = t * UNROLL + u
            q, q1, q2, q3 = u, (u + 1) % 4, (u + 2) % 4, (u + 3) % 4
            r = lax.rem(k, 3)
            r1 = lax.rem(k + 1, 3)
            r2 = lax.rem(k + 2, 3)
            gs_issue = (gsemA, gsemB)[u % 2]
            gs_wait = (gsemA, gsemB)[(u + 1) % 2]

            def chunk(do_pf, do_g2, do_w1):
                if do_pf:
                    dpf = pltpu.async_copy(
                        idx_hbm.at[base + k + 3], idx_v.at[q3], isem
                    )
                if do_g2:
                    pltpu.async_copy(
                        h_hbm.at[idx_v.at[q2, 0]], rows_v.at[r2], gs_issue
                    )
                pltpu.sync_copy(rows_v.at[r], acc_sh.at[idx_v.at[q, 1]], add=True)
                if do_w1:
                    pltpu.make_async_copy(
                        h_hbm.at[idx_v.at[q1, 0]], rows_v.at[r1], gs_wait
                    ).wait()
                if do_pf:
                    dpf.wait()

            if u == 0:
                chunk(True, True, True)
            else:
                tails = {1: (False, True, True), 2: (False, False, True),
                         3: (False, False, False)}[u]
                pl.when(t < NITER - 1)(lambda: chunk(True, True, True))
                pl.when(t == NITER - 1)(lambda: chunk(*tails))
        return carry

    lax.fori_loop(0, NITER, body, 0)

    plsc.subcore_barrier()

    @pl.when(s < NS - 1)
    def _():
        pltpu.sync_copy(
            acc_sh.at[pl.ds(s * RPT, RPT)], out_hbm.at[c, pl.ds(s * RPT, RPT)]
        )

    @pl.when(s == NS - 1)
    def _():
        pltpu.sync_copy(
            acc_sh.at[pl.ds((NS - 1) * RPT, RPT_LAST)],
            out_hbm.at[c, pl.ds((NS - 1) * RPT, RPT_LAST)],
        )


_RB = 1000


def _mm0_body(x_ref, w_ref, u_ref):
    u_ref[...] = jnp.dot(x_ref[...], w_ref[...], preferred_element_type=jnp.float32)


def _scale_body(u_ref, dg_ref, h_ref, dinv_ref):
    dinv = lax.rsqrt(dg_ref[:, 0:1] + dg_ref[:, 1:2])
    h_ref[...] = u_ref[...] * dinv
    dinv_ref[...] = dinv


def _mid_body(acc_ref, hp_ref, dinv_ref, b_ref, w_ref, out_ref):
    z = (
        dinv_ref[...] * (acc_ref[0] + acc_ref[1] - hp_ref[...]) + b_ref[...]
    )
    z = jnp.maximum(z, 0.0)
    out_ref[...] = (
        jnp.dot(z, w_ref[...], preferred_element_type=jnp.float32) * dinv_ref[...]
    )


def _fin_body(acc_ref, hp_ref, dinv_ref, b_ref, out_ref):
    out_ref[...] = (
        dinv_ref[...] * (acc_ref[0] + acc_ref[1] - hp_ref[...]) + b_ref[...]
    )


def _row_spec(width):
    return pl.BlockSpec((_RB, width), lambda i: (i, 0))


def _acc_spec():
    return pl.BlockSpec((2, _RB, D), lambda i: (0, i, 0))


def _const_spec(shape):
    return pl.BlockSpec(shape, lambda i: (0, 0))


_mm0 = pl.pallas_call(
    _mm0_body,
    grid=(N // _RB,),
    in_specs=[_row_spec(D), _const_spec((D, D))],
    out_specs=_row_spec(D),
    out_shape=jax.ShapeDtypeStruct((N, D), jnp.float32),
)

_scale = pl.pallas_call(
    _scale_body,
    grid=(N // _RB,),
    in_specs=[_row_spec(D), _row_spec(2)],
    out_specs=[_row_spec(D), _row_spec(1)],
    out_shape=[
        jax.ShapeDtypeStruct((N, D), jnp.float32),
        jax.ShapeDtypeStruct((N, 1), jnp.float32),
    ],
)

_mid = pl.pallas_call(
    _mid_body,
    grid=(N // _RB,),
    in_specs=[
        _acc_spec(), _row_spec(D), _row_spec(1),
        _const_spec((1, D)), _const_spec((D, D)),
    ],
    out_specs=_row_spec(D),
    out_shape=jax.ShapeDtypeStruct((N, D), jnp.float32),
)

_fin = pl.pallas_call(
    _fin_body,
    grid=(N // _RB,),
    in_specs=[
        _acc_spec(), _row_spec(D), _row_spec(1), _const_spec((1, D)),
    ],
    out_specs=_row_spec(D),
    out_shape=jax.ShapeDtypeStruct((N, D), jnp.float32),
)


def kernel(x, edge_index, W1, b1, W2, b2):
    ei = edge_index.astype(jnp.int32)
    idx3 = ei.reshape(2, NCHT, CH).transpose(1, 0, 2)

    deg2 = _deg_kernel(ei)
    u1 = _mm0(x, W1)

    h1p, dinv = _scale(u1, deg2.T)
    acc1 = _edge_kernel(h1p, idx3)
    h2p = _mid(acc1, h1p, dinv, b1.reshape(1, D), W2)
    acc2 = _edge_kernel(h2p, idx3)
    return _fin(acc2, h2p, dinv, b2.reshape(1, D))


# speedup vs baseline: 1.1264x; 1.0346x over previous
import functools

import jax
import jax.numpy as jnp
from jax import lax
from jax.experimental import pallas as pl
from jax.experimental.pallas import tpu as pltpu
from jax.experimental.pallas import tpu_sc as plsc

N = 10000
E = 320000
D = 128
NC = 2
NS = 16
NW = NC * NS
CH = 125
NCHT = E // CH
NCHUNK = NCHT // NW
UNROLL = 4
NITER = NCHUNK // UNROLL
RPT = 632
RPT_LAST = N - (NS - 1) * RPT
N2 = 10240
DPT = N2 // NS

_mesh = plsc.VectorSubcoreMesh(
    core_axis_name="c", subcore_axis_name="s", num_cores=NC, num_subcores=NS
)


CHD = 128
NCHD_TOT = E // CHD
NCHD = (NCHD_TOT + NW - 1) // NW


@functools.partial(
    pl.kernel,
    out_type=jax.ShapeDtypeStruct((NC, N2), jnp.float32),
    mesh=_mesh,
    compiler_params=pltpu.CompilerParams(needs_layout_passes=False),
    scratch_types=[
        pltpu.VMEM((N2,), jnp.float32),
        pltpu.VMEM((4, CHD), jnp.int32),
        pltpu.VMEM((NS, DPT), jnp.float32),
        pltpu.VMEM_SHARED((NS, N2), jnp.float32),
        pltpu.SemaphoreType.DMA,
        pltpu.SemaphoreType.DMA,
        pltpu.SemaphoreType.DMA,
        pltpu.SemaphoreType.DMA,
    ],
)
def _deg_kernel(ei_hbm, out_hbm, hist_v, idx_v, mrg_v, slab_sh, i0, i1, i2, i3):
    c = lax.axis_index("c")
    s = lax.axis_index("s")
    w = c * NS + s
    isems = (i0, i1, i2, i3)

    def zero(i, carry):
        for u in range(4):
            hist_v[pl.ds(i * 64 + u * 16, 16)] = jnp.zeros((16,), jnp.float32)
        return carry

    lax.fori_loop(0, N2 // 64, zero, 0)

    for q in range(3):

        def prime(q=q):
            pltpu.async_copy(
                ei_hbm.at[1, pl.ds((w + q * NW) * CHD, CHD)], idx_v.at[q], isems[q]
            )

        pl.when(w + q * NW < NCHD_TOT)(prime)

    ones16 = jnp.full((16,), 1.0, jnp.float32)

    def accum(q):
        for i in range(CHD // 16):
            idx16 = idx_v[q, pl.ds(i * 16, 16)]
            plsc.addupdate_scatter(hist_v, [idx16], ones16)

    def body(t, carry):
        for u in range(4):
            k = t * 4 + u
            q3 = (u + 3) % 4

            def sub(k=k, u=u, q3=q3):
                pltpu.make_async_copy(
                    ei_hbm.at[1, pl.ds(0, CHD)], idx_v.at[u], isems[u]
                ).wait()

                @pl.when(w + (k + 3) * NW < NCHD_TOT)
                def _():
                    pltpu.async_copy(
                        ei_hbm.at[1, pl.ds((w + (k + 3) * NW) * CHD, CHD)],
                        idx_v.at[q3],
                        isems[q3],
                    )

                accum(u)

            pl.when(w + k * NW < NCHD_TOT)(sub)
        return carry

    lax.fori_loop(0, (NCHD + 3) // 4, body, 0)

    pltpu.sync_copy(hist_v, slab_sh.at[s])
    plsc.subcore_barrier()
    pltpu.sync_copy(slab_sh.at[:, pl.ds(s * DPT, DPT)], mrg_v)

    def merge(i, carry):
        acc = jnp.full((16,), 0.5, jnp.float32)
        for t in range(NS):
            acc = acc + mrg_v[t, pl.ds(i * 16, 16)]
        hist_v[pl.ds(i * 16, 16)] = acc
        return carry

    lax.fori_loop(0, DPT // 16, merge, 0)
    pltpu.sync_copy(
        hist_v.at[pl.ds(0, DPT)], out_hbm.at[c, pl.ds(s * DPT, DPT)]
    )


@functools.partial(
    pl.kernel,
    out_type=jax.ShapeDtypeStruct((NC, N, D), jnp.float32),
    mesh=_mesh,
    scratch_types=[
        pltpu.VMEM((UNROLL, 2, CH), jnp.int32),
        pltpu.VMEM((3, CH, D), jnp.float32),
        pltpu.VMEM_SHARED((N, D), jnp.float32),
        pltpu.SemaphoreType.DMA,
        pltpu.SemaphoreType.DMA,
        pltpu.SemaphoreType.DMA,
        pltpu.SemaphoreType.DMA,
    ],
)
def _edge_kernel(
    h_hbm, z_hbm, idx_hbm, out_hbm, idx_v, rows_v, acc_sh, isem, gsemA, gsemB, asem
):
    c = lax.axis_index("c")
    s = lax.axis_index("s")
    w = c * NS + s
    base = w * NCHUNK

    def prologue(row0, nrows):
        def from_h():
            pltpu.async_copy(
                h_hbm.at[pl.ds(row0, nrows)], acc_sh.at[pl.ds(row0, nrows)], asem
            )

        def from_z():
            pltpu.async_copy(
                z_hbm.at[pl.ds(row0, nrows)], acc_sh.at[pl.ds(row0, nrows)], asem
            )

        pl.when(c == 0)(from_h)
        pl.when(c != 0)(from_z)
        ainit = pltpu.make_async_copy(
            h_hbm.at[pl.ds(row0, nrows)], acc_sh.at[pl.ds(row0, nrows)], asem
        )
        ip = pltpu.async_copy(
            idx_hbm.at[pl.ds(base, 3)], idx_v.at[pl.ds(0, 3)], isem
        )
        ip.wait()
        g0 = pltpu.async_copy(h_hbm.at[idx_v.at[0, 0]], rows_v.at[0], gsemA)
        pltpu.async_copy(h_hbm.at[idx_v.at[1, 0]], rows_v.at[1], gsemB)
        g0.wait()
        ainit.wait()


    pl.when(s < NS - 1)(lambda: prologue(s * RPT, RPT))
    pl.when(s == NS - 1)(lambda: prologue((NS - 1) * RPT, RPT_LAST))

    plsc.subcore_barrier()

    def body(t, carry):
        for u in range(UNROLL):
            k = t * UNROLL + u
            q, q1, q2, q3 = u, (u + 1) % 4, (u + 2) % 4, (u + 3) % 4
            r = lax.rem(k, 3)
            r1 = lax.rem(k + 1, 3)
            r2 = lax.rem(k + 2, 3)
            gs_issue = (gsemA, gsemB)[u % 2]
            gs_wait = (gsemA, gsemB)[(u + 1) % 2]

            def chunk(do_pf, do_g2, do_w1):
                if do_pf:
                    dpf = pltpu.async_copy(
                        idx_hbm.at[base + k + 3], idx_v.at[q3], isem
                    )
                if do_g2:
                    pltpu.async_copy(
                        h_hbm.at[idx_v.at[q2, 0]], rows_v.at[r2], gs_issue
                    )
                pltpu.sync_copy(rows_v.at[r], acc_sh.at[idx_v.at[q, 1]], add=True)
                if do_w1:
                    pltpu.make_async_copy(
                        h_hbm.at[idx_v.at[q1, 0]], rows_v.at[r1], gs_wait
                    ).wait()
                if do_pf:
                    dpf.wait()

            if u == 0:
                chunk(True, True, True)
            else:
                tails = {1: (False, True, True), 2: (False, False, True),
                         3: (False, False, False)}[u]
                pl.when(t < NITER - 1)(lambda: chunk(True, True, True))
                pl.when(t == NITER - 1)(lambda: chunk(*tails))
        return carry

    lax.fori_loop(0, NITER, body, 0)

    plsc.subcore_barrier()

    @pl.when(s < NS - 1)
    def _():
        pltpu.sync_copy(
            acc_sh.at[pl.ds(s * RPT, RPT)], out_hbm.at[c, pl.ds(s * RPT, RPT)]
        )

    @pl.when(s == NS - 1)
    def _():
        pltpu.sync_copy(
            acc_sh.at[pl.ds((NS - 1) * RPT, RPT_LAST)],
            out_hbm.at[c, pl.ds((NS - 1) * RPT, RPT_LAST)],
        )


_RB = 2000


def _mm0_body(x_ref, w_ref, u_ref):
    u_ref[...] = jnp.dot(x_ref[...], w_ref[...], preferred_element_type=jnp.float32)


def _scale_body(u_ref, dg_ref, h_ref, dinv_ref):
    dinv = lax.rsqrt(dg_ref[:, 0:1] + dg_ref[:, 1:2])
    h_ref[...] = u_ref[...] * dinv
    dinv_ref[...] = dinv


def _mid_body(acc_ref, dinv_ref, b_ref, w_ref, out_ref):
    z = dinv_ref[...] * (acc_ref[0] + acc_ref[1]) + b_ref[...]
    z = jnp.maximum(z, 0.0)
    out_ref[...] = (
        jnp.dot(z, w_ref[...], preferred_element_type=jnp.float32) * dinv_ref[...]
    )


def _fin_body(acc_ref, dinv_ref, b_ref, out_ref):
    out_ref[...] = dinv_ref[...] * (acc_ref[0] + acc_ref[1]) + b_ref[...]


def _row_spec(width):
    return pl.BlockSpec((_RB, width), lambda i: (i, 0))


def _acc_spec():
    return pl.BlockSpec((2, _RB, D), lambda i: (0, i, 0))


def _const_spec(shape):
    return pl.BlockSpec(shape, lambda i: (0, 0))


_mm0 = pl.pallas_call(
    _mm0_body,
    grid=(N // _RB,),
    in_specs=[_row_spec(D), _const_spec((D, D))],
    out_specs=_row_spec(D),
    out_shape=jax.ShapeDtypeStruct((N, D), jnp.float32),
)

_scale = pl.pallas_call(
    _scale_body,
    grid=(N // _RB,),
    in_specs=[_row_spec(D), _row_spec(2)],
    out_specs=[_row_spec(D), _row_spec(1)],
    out_shape=[
        jax.ShapeDtypeStruct((N, D), jnp.float32),
        jax.ShapeDtypeStruct((N, 1), jnp.float32),
    ],
)

_mid = pl.pallas_call(
    _mid_body,
    grid=(N // _RB,),
    in_specs=[
        _acc_spec(), _row_spec(1), _const_spec((1, D)), _const_spec((D, D)),
    ],
    out_specs=_row_spec(D),
    out_shape=jax.ShapeDtypeStruct((N, D), jnp.float32),
)

_fin = pl.pallas_call(
    _fin_body,
    grid=(N // _RB,),
    in_specs=[_acc_spec(), _row_spec(1), _const_spec((1, D))],
    out_specs=_row_spec(D),
    out_shape=jax.ShapeDtypeStruct((N, D), jnp.float32),
)


def kernel(x, edge_index, W1, b1, W2, b2):
    ei = edge_index.astype(jnp.int32)
    idx3 = ei.reshape(2, NCHT, CH).transpose(1, 0, 2)

    deg2 = _deg_kernel(ei)
    u1 = _mm0(x, W1)

    zeros = jnp.zeros((N, D), jnp.float32)
    h1p, dinv = _scale(u1, deg2.T)
    acc1 = _edge_kernel(h1p, zeros, idx3)
    h2p = _mid(acc1, dinv, b1.reshape(1, D), W2)
    acc2 = _edge_kernel(h2p, zeros, idx3)
    return _fin(acc2, dinv, b2.reshape(1, D))


# speedup vs baseline: 1.1300x; 1.0032x over previous
import functools

import jax
import jax.numpy as jnp
from jax import lax
from jax.experimental import pallas as pl
from jax.experimental.pallas import tpu as pltpu
from jax.experimental.pallas import tpu_sc as plsc

N = 10000
E = 320000
D = 128
NC = 2
NS = 16
NW = NC * NS
CH = 125
NCHT = E // CH
NCHUNK = NCHT // NW
UNROLL = 4
NITER = NCHUNK // UNROLL
RPT = 632
RPT_LAST = N - (NS - 1) * RPT
N2 = 10240
DPT = N2 // NS

_mesh = plsc.VectorSubcoreMesh(
    core_axis_name="c", subcore_axis_name="s", num_cores=NC, num_subcores=NS
)


CHD = 128
NCHD_TOT = E // CHD
NCHD = (NCHD_TOT + NW - 1) // NW


@functools.partial(
    pl.kernel,
    out_type=jax.ShapeDtypeStruct((NC, N2), jnp.float32),
    mesh=_mesh,
    compiler_params=pltpu.CompilerParams(needs_layout_passes=False),
    scratch_types=[
        pltpu.VMEM((N2,), jnp.float32),
        pltpu.VMEM((4, CHD), jnp.int32),
        pltpu.VMEM((NS, DPT), jnp.float32),
        pltpu.VMEM_SHARED((NS, N2), jnp.float32),
        pltpu.SemaphoreType.DMA,
        pltpu.SemaphoreType.DMA,
        pltpu.SemaphoreType.DMA,
        pltpu.SemaphoreType.DMA,
    ],
)
def _deg_kernel(ei_hbm, out_hbm, hist_v, idx_v, mrg_v, slab_sh, i0, i1, i2, i3):
    c = lax.axis_index("c")
    s = lax.axis_index("s")
    w = c * NS + s
    isems = (i0, i1, i2, i3)

    def zero(i, carry):
        for u in range(4):
            hist_v[pl.ds(i * 64 + u * 16, 16)] = jnp.zeros((16,), jnp.float32)
        return carry

    lax.fori_loop(0, N2 // 64, zero, 0)

    for q in range(3):

        def prime(q=q):
            pltpu.async_copy(
                ei_hbm.at[1, pl.ds((w + q * NW) * CHD, CHD)], idx_v.at[q], isems[q]
            )

        pl.when(w + q * NW < NCHD_TOT)(prime)

    ones16 = jnp.full((16,), 1.0, jnp.float32)

    def accum(q):
        for i in range(CHD // 16):
            idx16 = idx_v[q, pl.ds(i * 16, 16)]
            plsc.addupdate_scatter(hist_v, [idx16], ones16)

    def body(t, carry):
        for u in range(4):
            k = t * 4 + u
            q3 = (u + 3) % 4

            def sub(k=k, u=u, q3=q3):
                pltpu.make_async_copy(
                    ei_hbm.at[1, pl.ds(0, CHD)], idx_v.at[u], isems[u]
                ).wait()

                @pl.when(w + (k + 3) * NW < NCHD_TOT)
                def _():
                    pltpu.async_copy(
                        ei_hbm.at[1, pl.ds((w + (k + 3) * NW) * CHD, CHD)],
                        idx_v.at[q3],
                        isems[q3],
                    )

                accum(u)

            pl.when(w + k * NW < NCHD_TOT)(sub)
        return carry

    lax.fori_loop(0, (NCHD + 3) // 4, body, 0)

    pltpu.sync_copy(hist_v, slab_sh.at[s])
    plsc.subcore_barrier()
    pltpu.sync_copy(slab_sh.at[:, pl.ds(s * DPT, DPT)], mrg_v)

    def merge(i, carry):
        acc = jnp.full((16,), 0.5, jnp.float32)
        for t in range(NS):
            acc = acc + mrg_v[t, pl.ds(i * 16, 16)]
        hist_v[pl.ds(i * 16, 16)] = acc
        return carry

    lax.fori_loop(0, DPT // 16, merge, 0)
    pltpu.sync_copy(
        hist_v.at[pl.ds(0, DPT)], out_hbm.at[c, pl.ds(s * DPT, DPT)]
    )


@functools.partial(
    pl.kernel,
    out_type=jax.ShapeDtypeStruct((NC, N, D), jnp.float32),
    mesh=_mesh,
    scratch_types=[
        pltpu.VMEM((UNROLL, 2, CH), jnp.int32),
        pltpu.VMEM((3, CH, D), jnp.float32),
        pltpu.VMEM_SHARED((N, D), jnp.float32),
        pltpu.SemaphoreType.DMA,
        pltpu.SemaphoreType.DMA,
        pltpu.SemaphoreType.DMA,
        pltpu.SemaphoreType.DMA,
    ],
)
def _edge_kernel(
    h_hbm, z_hbm, idx_hbm, out_hbm, idx_v, rows_v, acc_sh, isem, gsemA, gsemB, asem
):
    c = lax.axis_index("c")
    s = lax.axis_index("s")
    w = c * NS + s
    base = w * NCHUNK

    def prologue(row0, nrows):
        def from_h():
            pltpu.async_copy(
                h_hbm.at[pl.ds(row0, nrows)], acc_sh.at[pl.ds(row0, nrows)], asem
            )

        def from_z():
            pltpu.async_copy(
                z_hbm.at[pl.ds(0, nrows)], acc_sh.at[pl.ds(row0, nrows)], asem
            )

        pl.when(c == 0)(from_h)
        pl.when(c != 0)(from_z)
        ainit = pltpu.make_async_copy(
            h_hbm.at[pl.ds(row0, nrows)], acc_sh.at[pl.ds(row0, nrows)], asem
        )
        ip = pltpu.async_copy(
            idx_hbm.at[pl.ds(base, 3)], idx_v.at[pl.ds(0, 3)], isem
        )
        ip.wait()
        g0 = pltpu.async_copy(h_hbm.at[idx_v.at[0, 0]], rows_v.at[0], gsemA)
        pltpu.async_copy(h_hbm.at[idx_v.at[1, 0]], rows_v.at[1], gsemB)
        g0.wait()
        ainit.wait()


    pl.when(s < NS - 1)(lambda: prologue(s * RPT, RPT))
    pl.when(s == NS - 1)(lambda: prologue((NS - 1) * RPT, RPT_LAST))

    plsc.subcore_barrier()

    def body(t, carry):
        for u in range(UNROLL):
            k = t * UNROLL + u
            q, q1, q2, q3 = u, (u + 1) % 4, (u + 2) % 4, (u + 3) % 4
            r = lax.rem(k, 3)
            r1 = lax.rem(k + 1, 3)
            r2 = lax.rem(k + 2, 3)
            gs_issue = (gsemA, gsemB)[u % 2]
            gs_wait = (gsemA, gsemB)[(u + 1) % 2]

            def chunk(do_pf, do_g2, do_w1):
                if do_pf:
                    dpf = pltpu.async_copy(
                        idx_hbm.at[base + k + 3], idx_v.at[q3], isem
                    )
                if do_g2:
                    pltpu.async_copy(
                        h_hbm.at[idx_v.at[q2, 0]], rows_v.at[r2], gs_issue
                    )
                pltpu.sync_copy(rows_v.at[r], acc_sh.at[idx_v.at[q, 1]], add=True)
                if do_w1:
                    pltpu.make_async_copy(
                        h_hbm.at[idx_v.at[q1, 0]], rows_v.at[r1], gs_wait
                    ).wait()
                if do_pf:
                    dpf.wait()

            if u == 0:
                chunk(True, True, True)
            else:
                tails = {1: (False, True, True), 2: (False, False, True),
                         3: (False, False, False)}[u]
                pl.when(t < NITER - 1)(lambda: chunk(True, True, True))
                pl.when(t == NITER - 1)(lambda: chunk(*tails))
        return carry

    lax.fori_loop(0, NITER, body, 0)

    plsc.subcore_barrier()

    @pl.when(s < NS - 1)
    def _():
        pltpu.sync_copy(
            acc_sh.at[pl.ds(s * RPT, RPT)], out_hbm.at[c, pl.ds(s * RPT, RPT)]
        )

    @pl.when(s == NS - 1)
    def _():
        pltpu.sync_copy(
            acc_sh.at[pl.ds((NS - 1) * RPT, RPT_LAST)],
            out_hbm.at[c, pl.ds((NS - 1) * RPT, RPT_LAST)],
        )


_RB = 2000


def _mm0_body(x_ref, w_ref, u_ref):
    u_ref[...] = jnp.dot(x_ref[...], w_ref[...], preferred_element_type=jnp.float32)


def _scale_body(u_ref, dg_ref, h_ref, dinv_ref):
    dinv = lax.rsqrt(dg_ref[:, 0:1] + dg_ref[:, 1:2])
    h_ref[...] = u_ref[...] * dinv
    dinv_ref[...] = dinv


def _mid_body(acc_ref, dinv_ref, b_ref, w_ref, out_ref):
    z = dinv_ref[...] * (acc_ref[0] + acc_ref[1]) + b_ref[...]
    z = jnp.maximum(z, 0.0)
    out_ref[...] = (
        jnp.dot(z, w_ref[...], preferred_element_type=jnp.float32) * dinv_ref[...]
    )


def _fin_body(acc_ref, dinv_ref, b_ref, out_ref):
    out_ref[...] = dinv_ref[...] * (acc_ref[0] + acc_ref[1]) + b_ref[...]


def _row_spec(width):
    return pl.BlockSpec((_RB, width), lambda i: (i, 0))


def _acc_spec():
    return pl.BlockSpec((2, _RB, D), lambda i: (0, i, 0))


def _const_spec(shape):
    return pl.BlockSpec(shape, lambda i: (0, 0))


_mm0 = pl.pallas_call(
    _mm0_body,
    grid=(N // _RB,),
    in_specs=[_row_spec(D), _const_spec((D, D))],
    out_specs=_row_spec(D),
    out_shape=jax.ShapeDtypeStruct((N, D), jnp.float32),
)

_scale = pl.pallas_call(
    _scale_body,
    grid=(N // _RB,),
    in_specs=[_row_spec(D), _row_spec(2)],
    out_specs=[_row_spec(D), _row_spec(1)],
    out_shape=[
        jax.ShapeDtypeStruct((N, D), jnp.float32),
        jax.ShapeDtypeStruct((N, 1), jnp.float32),
    ],
)

_mid = pl.pallas_call(
    _mid_body,
    grid=(N // _RB,),
    in_specs=[
        _acc_spec(), _row_spec(1), _const_spec((1, D)), _const_spec((D, D)),
    ],
    out_specs=_row_spec(D),
    out_shape=jax.ShapeDtypeStruct((N, D), jnp.float32),
)

_fin = pl.pallas_call(
    _fin_body,
    grid=(N // _RB,),
    in_specs=[_acc_spec(), _row_spec(1), _const_spec((1, D))],
    out_specs=_row_spec(D),
    out_shape=jax.ShapeDtypeStruct((N, D), jnp.float32),
)


def kernel(x, edge_index, W1, b1, W2, b2):
    ei = edge_index.astype(jnp.int32)
    idx3 = ei.reshape(2, NCHT, CH).transpose(1, 0, 2)

    deg2 = _deg_kernel(ei)
    u1 = _mm0(x, W1)

    zeros = jnp.zeros((RPT, D), jnp.float32)
    h1p, dinv = _scale(u1, deg2.T)
    acc1 = _edge_kernel(h1p, zeros, idx3)
    h2p = _mid(acc1, dinv, b1.reshape(1, D), W2)
    acc2 = _edge_kernel(h2p, zeros, idx3)
    return _fin(acc2, dinv, b2.reshape(1, D))
